# baseline (speedup 1.0000x reference)
import functools

import jax
import jax.numpy as jnp
from jax import lax
from jax.experimental import pallas as pl
from jax.experimental.pallas import tpu as pltpu
from jax.experimental.pallas import tpu_sc as plsc

_NUM_FIELDS = 26
_VOCAB = 100000
_EMBED_DIM = 32
_BATCH = 4096

_INFO = plsc.get_sparse_core_info()
_NC = _INFO.num_cores
_NS = _INFO.num_subcores
_NW = _NC * _NS
_BPW = _BATCH // _NW


def _make_gather():
    mesh = plsc.VectorSubcoreMesh(core_axis_name="c", subcore_axis_name="s")

    @functools.partial(
        pl.kernel,
        mesh=mesh,
        out_type=jax.ShapeDtypeStruct((_BATCH, 1, _NUM_FIELDS * _EMBED_DIM),
                                      jnp.float32),
        scratch_types=[
            pltpu.VMEM((_NUM_FIELDS, _BPW), jnp.int32),
            pltpu.VMEM((_NUM_FIELDS, _BPW, _EMBED_DIM), jnp.float32),
            pltpu.SemaphoreType.DMA,
        ],
        compiler_params=pltpu.CompilerParams(use_tc_tiling_on_sc=False),
    )
    def gather_kernel(idx_hbm, tables_hbm, out_hbm, idx_v, rows_v, sem):
        wid = lax.axis_index("s") * _NC + lax.axis_index("c")
        base = wid * _BPW
        pltpu.sync_copy(idx_hbm.at[:, pl.ds(base, _BPW)], idx_v)

        def fire(f, carry):
            pltpu.async_copy(tables_hbm.at[f].at[idx_v.at[f]], rows_v.at[f],
                             sem)
            return carry

        lax.fori_loop(0, _NUM_FIELDS, fire, 0)

        def drain(f, carry):
            pltpu.make_async_copy(tables_hbm.at[f].at[idx_v.at[f]],
                                  rows_v.at[f], sem).wait()
            pltpu.sync_copy(rows_v.at[f],
                            out_hbm.at[pl.ds(base, _BPW), 0,
                                       pl.ds(f * _EMBED_DIM, _EMBED_DIM)])
            return carry

        lax.fori_loop(0, _NUM_FIELDS, drain, 0)

    return gather_kernel


_GATHER = _make_gather()


def kernel(inputs, tables):
    idx_t = inputs.astype(jnp.int32).T
    return _GATHER(idx_t, tables)


# speedup vs baseline: 5.9402x; 5.9402x over previous
import functools

import jax
import jax.numpy as jnp
from jax import lax
from jax.experimental import pallas as pl
from jax.experimental.pallas import tpu as pltpu
from jax.experimental.pallas import tpu_sc as plsc

_NF = 26
_V = 100000
_D = 32
_B = 4096

_INFO = plsc.get_sparse_core_info()
_NC = _INFO.num_cores
_NS = _INFO.num_subcores
_NW = _NC * _NS
_CPW = _NF * _D // _NW


def _make_kernel():
    mesh = plsc.VectorSubcoreMesh(core_axis_name="c", subcore_axis_name="s")

    @functools.partial(
        pl.kernel,
        mesh=mesh,
        out_type=jax.ShapeDtypeStruct((_NF * _D, _B), jnp.float32),
        scratch_types=[
            pltpu.VMEM((_V,), jnp.float32),
            pltpu.VMEM((_B,), jnp.int32),
            pltpu.VMEM((_B,), jnp.float32),
        ],
        compiler_params=pltpu.CompilerParams(use_tc_tiling_on_sc=True,
                                             needs_layout_passes=False),
    )
    def col_kernel(idx_hbm, tab_hbm, out_hbm, col_v, idx_v, row_v):
        w = lax.axis_index("s") * _NC + lax.axis_index("c")

        def pair(j, carry):
            c = w * _CPW + j
            f = c // _D
            d = c % _D
            pltpu.sync_copy(idx_hbm.at[f], idx_v)
            pltpu.sync_copy(tab_hbm.at[f, d], col_v)

            def gat(i, c2):
                sl = pl.ds(i * 16, 16)
                row_v[sl] = plsc.load_gather(col_v, [idx_v[sl]])
                return c2

            lax.fori_loop(0, _B // 16, gat, 0)
            pltpu.sync_copy(row_v, out_hbm.at[c])
            return carry

        lax.fori_loop(0, _CPW, pair, 0)

    return col_kernel


_KERNEL = _make_kernel()


def kernel(inputs, tables):
    tab_t = tables.transpose(0, 2, 1)
    idx_t = inputs.astype(jnp.int32).T
    out_t = _KERNEL(idx_t, tab_t)
    return out_t.T.reshape(_B, 1, _NF * _D)


# speedup vs baseline: 7.8043x; 1.3138x over previous
import functools

import jax
import jax.numpy as jnp
from jax import lax
from jax.experimental import pallas as pl
from jax.experimental.pallas import tpu as pltpu
from jax.experimental.pallas import tpu_sc as plsc

_NF = 26
_V = 100000
_D = 32
_B = 4096
_H = 49920
_H2 = _V - _H

_INFO = plsc.get_sparse_core_info()
_NC = _INFO.num_cores
_NS = _INFO.num_subcores
_NW = _NC * _NS
_CPW = _NF * _D // _NW
_UNROLL = 4


def _make_kernel():
    mesh = plsc.VectorSubcoreMesh(core_axis_name="c", subcore_axis_name="s")

    @functools.partial(
        pl.kernel,
        mesh=mesh,
        out_type=jax.ShapeDtypeStruct((_NF * _D, _B), jnp.float32),
        scratch_types=[
            pltpu.VMEM((_H,), jnp.float32),
            pltpu.VMEM((_H2,), jnp.float32),
            pltpu.VMEM((_B,), jnp.int32),
            pltpu.VMEM((_B,), jnp.float32),
            pltpu.SemaphoreType.DMA,
            pltpu.SemaphoreType.DMA,
        ],
        compiler_params=pltpu.CompilerParams(use_tc_tiling_on_sc=True,
                                             needs_layout_passes=False),
    )
    def col_kernel(idx_hbm, tab_hbm, out_hbm, col_lo, col_hi, idx_v, row_v,
                   sem_lo, sem_hi):
        w = lax.axis_index("s") * _NC + lax.axis_index("c")
        c0 = w * _CPW

        pltpu.async_copy(tab_hbm.at[c0 // _D, c0 % _D, pl.ds(0, _H)],
                         col_lo, sem_lo)
        pltpu.async_copy(tab_hbm.at[c0 // _D, c0 % _D, pl.ds(_H, _H2)],
                         col_hi, sem_hi)

        def pair(j, f_prev):
            c = c0 + j
            f = c // _D
            d = c % _D

            @pl.when(f != f_prev)
            def _():
                pltpu.sync_copy(idx_hbm.at[f], idx_v)

            pltpu.make_async_copy(tab_hbm.at[f, d, pl.ds(0, _H)],
                                  col_lo, sem_lo).wait()

            def gat_lo(i, c2):
                for u in range(_UNROLL):
                    sl = pl.ds((i * _UNROLL + u) * 16, 16)
                    ix = idx_v[sl]
                    m = ix < _H
                    g = plsc.load_gather(col_lo, [ix], mask=m)
                    row_v[sl] = jnp.where(m, g, 0.0)
                return c2

            lax.fori_loop(0, _B // (16 * _UNROLL), gat_lo, 0)

            @pl.when(j < _CPW - 1)
            def _():
                c1 = c + 1
                pltpu.async_copy(
                    tab_hbm.at[c1 // _D, c1 % _D, pl.ds(0, _H)],
                    col_lo, sem_lo)

            pltpu.make_async_copy(tab_hbm.at[f, d, pl.ds(_H, _H2)],
                                  col_hi, sem_hi).wait()

            def gat_hi(i, c2):
                for u in range(_UNROLL):
                    sl = pl.ds((i * _UNROLL + u) * 16, 16)
                    ix = idx_v[sl]
                    m = ix >= _H
                    g = plsc.load_gather(col_hi, [ix - _H], mask=m)
                    row_v[sl] = jnp.where(m, g, row_v[sl])
                return c2

            lax.fori_loop(0, _B // (16 * _UNROLL), gat_hi, 0)

            @pl.when(j < _CPW - 1)
            def _():
                c1 = c + 1
                pltpu.async_copy(
                    tab_hbm.at[c1 // _D, c1 % _D, pl.ds(_H, _H2)],
                    col_hi, sem_hi)

            pltpu.sync_copy(row_v, out_hbm.at[c])
            return f

        lax.fori_loop(0, _CPW, pair, jnp.int32(-1))

    return col_kernel


_KERNEL = _make_kernel()


def kernel(inputs, tables):
    tab_t = tables.transpose(0, 2, 1)
    idx_t = inputs.astype(jnp.int32).T
    out_t = _KERNEL(idx_t, tab_t)
    return out_t.T.reshape(_B, 1, _NF * _D)


# speedup vs baseline: 7.9330x; 1.0165x over previous
import functools

import jax
import jax.numpy as jnp
from jax import lax
from jax.experimental import pallas as pl
from jax.experimental.pallas import tpu as pltpu
from jax.experimental.pallas import tpu_sc as plsc

_NF = 26
_V = 100000
_D = 32
_B = 4096

_Q = 24960
_SPLIT = ((0, _Q), (_Q, _Q), (2 * _Q, _Q), (3 * _Q, _V - 3 * _Q))

_INFO = plsc.get_sparse_core_info()
_NC = _INFO.num_cores
_NS = _INFO.num_subcores
_NW = _NC * _NS
_CPW = _NF * _D // _NW
_UNROLL = 4


def _make_kernel():
    mesh = plsc.VectorSubcoreMesh(core_axis_name="c", subcore_axis_name="s")

    @functools.partial(
        pl.kernel,
        mesh=mesh,
        out_type=jax.ShapeDtypeStruct((_NF * _D, _B), jnp.float32),
        scratch_types=[
            pltpu.VMEM((_SPLIT[0][1],), jnp.float32),
            pltpu.VMEM((_SPLIT[1][1],), jnp.float32),
            pltpu.VMEM((_SPLIT[2][1],), jnp.float32),
            pltpu.VMEM((_SPLIT[3][1],), jnp.float32),
            pltpu.VMEM((_B,), jnp.int32),
            pltpu.VMEM((_B,), jnp.float32),
            pltpu.SemaphoreType.DMA,
            pltpu.SemaphoreType.DMA,
            pltpu.SemaphoreType.DMA,
            pltpu.SemaphoreType.DMA,
        ],
        compiler_params=pltpu.CompilerParams(use_tc_tiling_on_sc=True,
                                             needs_layout_passes=False),
    )
    def col_kernel(idx_hbm, tab_hbm, out_hbm, q0, q1, q2, q3, idx_v, row_v,
                   s0, s1, s2, s3):
        bufs = (q0, q1, q2, q3)
        sems = (s0, s1, s2, s3)
        w = lax.axis_index("s") * _NC + lax.axis_index("c")
        c0 = w * _CPW

        for q, (off, ln) in enumerate(_SPLIT):
            pltpu.async_copy(tab_hbm.at[c0 // _D, c0 % _D, pl.ds(off, ln)],
                             bufs[q], sems[q])

        def pair(j, f_prev):
            c = c0 + j
            f = c // _D
            d = c % _D

            @pl.when(f != f_prev)
            def _():
                pltpu.sync_copy(idx_hbm.at[f], idx_v)

            for q, (off, ln) in enumerate(_SPLIT):
                pltpu.make_async_copy(tab_hbm.at[f, d, pl.ds(off, ln)],
                                      bufs[q], sems[q]).wait()
                buf = bufs[q]

                def gat(i, c2, q=q, off=off, ln=ln, buf=buf):
                    for u in range(_UNROLL):
                        sl = pl.ds((i * _UNROLL + u) * 16, 16)
                        ix = idx_v[sl] - off
                        m = (ix >= 0) & (ix < ln)
                        g = plsc.load_gather(buf, [ix], mask=m)
                        if q == 0:
                            row_v[sl] = jnp.where(m, g, 0.0)
                        else:
                            row_v[sl] = jnp.where(m, g, row_v[sl])
                    return c2

                lax.fori_loop(0, _B // (16 * _UNROLL), gat, 0)

                @pl.when(j < _CPW - 1)
                def _(q=q, off=off, ln=ln):
                    c1 = c + 1
                    pltpu.async_copy(
                        tab_hbm.at[c1 // _D, c1 % _D, pl.ds(off, ln)],
                        bufs[q], sems[q])

            pltpu.sync_copy(row_v, out_hbm.at[c])
            return f

        lax.fori_loop(0, _CPW, pair, jnp.int32(-1))

    return col_kernel


_KERNEL = _make_kernel()


def kernel(inputs, tables):
    tab_t = tables.transpose(0, 2, 1)
    idx_t = inputs.astype(jnp.int32).T
    out_t = _KERNEL(idx_t, tab_t)
    return out_t.T.reshape(_B, 1, _NF * _D)
